# baseline (speedup 1.0000x reference)
import functools

import jax
import jax.numpy as jnp
from jax import lax
from jax.experimental import pallas as pl
from jax.experimental.pallas import tpu as pltpu
from jax.experimental.pallas import tpu_sc as plsc

B, L, D = 4, 4096, 1024
LANES = 16
NC, NS = 2, 16
NW = NC * NS
W_PER_B = NW // B
ROWS_PER_W = L // W_PER_B
CHUNK = LANES
NCHUNK = ROWS_PER_W // CHUNK
HALF = CHUNK // 2
NX = 2
NP = 4


def _pe_add(x, offset_bcast, pe):
    mesh = plsc.VectorSubcoreMesh(core_axis_name="c", subcore_axis_name="s")

    buf = lambda: pltpu.VMEM((CHUNK, D), jnp.float32)
    obuf_t = lambda: pltpu.VMEM((HALF, D), jnp.float32)
    sem = pltpu.SemaphoreType.DMA

    @functools.partial(
        pl.kernel,
        mesh=mesh,
        out_type=jax.ShapeDtypeStruct((B, L, D), jnp.float32),
        scratch_types=[pltpu.VMEM((LANES,), jnp.int32)]
        + [buf() for _ in range(NX + NP)]
        + [obuf_t() for _ in range(2)]
        + [sem for _ in range(NX + NP + 2)],
    )
    def k(x_hbm, off_hbm, pe_hbm, out_hbm, off_v,
          xb0, xb1, pb0, pb1, pb2, pb3, ob0, ob1,
          xs0, xs1, ps0, ps1, ps2, ps3, os0, os1):
        xbuf, pbuf, obuf = (xb0, xb1), (pb0, pb1, pb2, pb3), (ob0, ob1)
        xsem, psem, osem = (xs0, xs1), (ps0, ps1, ps2, ps3), (os0, os1)

        wid = lax.axis_index("s") * NC + lax.axis_index("c")
        b = wid // W_PER_B
        l_base = (wid % W_PER_B) * ROWS_PER_W

        pltpu.sync_copy(off_hbm.at[wid], off_v)
        offv = off_v[...]
        iot = lax.iota(jnp.int32, LANES)

        def start_x(s, c):
            pltpu.make_async_copy(
                x_hbm.at[b, pl.ds(l_base + c * CHUNK, CHUNK)],
                xbuf[s], xsem[s]).start()

        def start_pe(s, c):
            idxv = jnp.maximum(iot + (l_base + c * CHUNK + 1) - offv, 0)
            pltpu.make_async_copy(pe_hbm.at[idxv], pbuf[s], psem[s]).start()

        def wait_in(sx, sp):
            pltpu.make_async_copy(
                x_hbm.at[b, pl.ds(0, CHUNK)], xbuf[sx], xsem[sx]).wait()
            pltpu.make_async_copy(
                pe_hbm.at[pl.ds(0, CHUNK)], pbuf[sp], psem[sp]).wait()

        def start_out_half(h, c, lo):
            l0 = l_base + c * CHUNK + lo
            pltpu.make_async_copy(
                obuf[h], out_hbm.at[b, pl.ds(l0, HALF)], osem[h]).start()

        def wait_out_half(h):
            pltpu.make_async_copy(
                obuf[h], out_hbm.at[b, pl.ds(0, HALF)], osem[h]).wait()

        for c in range(NX):
            start_x(c, c)
        for c in range(NP):
            start_pe(c, c)

        def group_body(g, carry):
            for j in range(NP):
                c = g * NP + j
                sx = j % NX
                sp = j % NP
                wait_in(sx, sp)
                for h in range(2):
                    if j == 0:
                        @pl.when(g > 0)
                        def _():
                            wait_out_half(h)
                    else:
                        wait_out_half(h)

                    def half_body(r, rc):
                        for cc in range(D // LANES):
                            sl = pl.ds(cc * LANES, LANES)
                            obuf[h][r, sl] = (xbuf[sx][r + h * HALF, sl]
                                              + pbuf[sp][r + h * HALF, sl])
                        return rc
                    lax.fori_loop(0, HALF, half_body, 0)

                    start_out_half(h, c, h * HALF)

                @pl.when(c + NX < NCHUNK)
                def _():
                    start_x(sx, c + NX)

                @pl.when(c + NP < NCHUNK)
                def _():
                    start_pe(sp, c + NP)
            return carry

        lax.fori_loop(0, NCHUNK // NP, group_body, 0)

        wait_out_half(0)
        wait_out_half(1)

    return k(x, offset_bcast, pe)


def kernel(x, offset, pe):
    off_bcast = jnp.broadcast_to(
        offset.reshape(B, 1, 1).astype(jnp.int32), (B, W_PER_B, LANES)
    ).reshape(NW, LANES)
    return _pe_add(x, off_bcast, pe)


# speedup vs baseline: 1.0806x; 1.0806x over previous
import functools

import jax
import jax.numpy as jnp
from jax import lax
from jax.experimental import pallas as pl
from jax.experimental.pallas import tpu as pltpu
from jax.experimental.pallas import tpu_sc as plsc

B, L, D = 4, 4096, 1024
LANES = 16
NC, NS = 2, 16
NW = NC * NS
W_PER_B = NW // B
ROWS_PER_W = L // W_PER_B
CHUNK = LANES
NCHUNK = ROWS_PER_W // CHUNK
NBUF = 2


def _pe_add(x, offset_bcast, pe):
    mesh = plsc.VectorSubcoreMesh(core_axis_name="c", subcore_axis_name="s")

    buf = lambda: pltpu.VMEM((CHUNK, D), jnp.float32)
    sem = pltpu.SemaphoreType.DMA

    @functools.partial(
        pl.kernel,
        mesh=mesh,
        out_type=jax.ShapeDtypeStruct((B, L, D), jnp.float32),
        scratch_types=[pltpu.VMEM((LANES,), jnp.int32)]
        + [buf() for _ in range(3 * NBUF)]
        + [sem for _ in range(3 * NBUF)],
    )
    def k(x_hbm, off_hbm, pe_hbm, out_hbm, off_v,
          xb0, xb1, pb0, pb1, ob0, ob1,
          xs0, xs1, ps0, ps1, os0, os1):
        xbuf, pbuf, obuf = (xb0, xb1), (pb0, pb1), (ob0, ob1)
        xsem, psem, osem = (xs0, xs1), (ps0, ps1), (os0, os1)

        wid = lax.axis_index("s") * NC + lax.axis_index("c")
        b = wid // W_PER_B
        l_base = (wid % W_PER_B) * ROWS_PER_W

        pltpu.sync_copy(off_hbm.at[wid], off_v)
        offv = off_v[...]
        iot = lax.iota(jnp.int32, LANES)

        def start_in(s, l0):
            pltpu.make_async_copy(
                x_hbm.at[b, pl.ds(l0, CHUNK)], xbuf[s], xsem[s]).start()
            idxv = jnp.maximum(iot + (l0 + 1) - offv, 0)
            pltpu.make_async_copy(pe_hbm.at[idxv], pbuf[s], psem[s]).start()

        def wait_in(s):
            pltpu.make_async_copy(
                x_hbm.at[b, pl.ds(0, CHUNK)], xbuf[s], xsem[s]).wait()
            pltpu.make_async_copy(
                pe_hbm.at[pl.ds(0, CHUNK)], pbuf[s], psem[s]).wait()

        for s in range(NBUF):
            start_in(s, l_base + s * CHUNK)

        def group_body(g, carry):
            for s in range(NBUF):
                c = g * NBUF + s
                l0 = l_base + c * CHUNK
                wait_in(s)

                @pl.when(g > 0)
                def _():
                    pltpu.make_async_copy(
                        obuf[s], out_hbm.at[b, pl.ds(l0, CHUNK)],
                        osem[s]).wait()

                def row_body(r, rc):
                    for cc in range(D // LANES):
                        sl = pl.ds(cc * LANES, LANES)
                        obuf[s][r, sl] = xbuf[s][r, sl] + pbuf[s][r, sl]
                    return rc
                lax.fori_loop(0, CHUNK, row_body, 0)

                pltpu.make_async_copy(
                    obuf[s], out_hbm.at[b, pl.ds(l0, CHUNK)], osem[s]).start()

                @pl.when(c + NBUF < NCHUNK)
                def _():
                    start_in(s, l0 + NBUF * CHUNK)
            return carry

        lax.fori_loop(0, NCHUNK // NBUF, group_body, 0)

        for s in range(NBUF):
            c = NCHUNK - NBUF + s
            l0 = l_base + c * CHUNK
            pltpu.make_async_copy(
                obuf[s], out_hbm.at[b, pl.ds(l0, CHUNK)], osem[s]).wait()

    return k(x, offset_bcast, pe)


def kernel(x, offset, pe):
    off_bcast = jnp.broadcast_to(
        offset.reshape(B, 1, 1).astype(jnp.int32), (B, W_PER_B, LANES)
    ).reshape(NW, LANES)
    return _pe_add(x, off_bcast, pe)
